# baseline (speedup 1.0000x reference)
import jax
import jax.numpy as jnp
from jax.experimental import pallas as pl
from jax.experimental.pallas import tpu as pltpu

_B = 1024
_P = 32
_H = 256
_V = 25000
_T = 4
_VT = 2048
_NVT = (_V + _VT - 1) // _VT
_NEG = float("-inf")
_NFIX = 256


def _bf16_val(x):
    u = jax.lax.bitcast_convert_type(x, jnp.int32)
    r = u + 0x7FFF + ((u >> 16) & 1)
    q = r & jnp.int32(-65536)
    return jax.lax.bitcast_convert_type(q, jnp.float32)


def _tok_kernel(prop_ref, w1_ref, b1_ref, w2_ref, b2_ref, wq_ref, bq_ref,
                sin_ref, tok_ref, gap_ref, fe_ref, feats_s, bestv_s,
                best2_s, besti_s):
    j = pl.program_id(0)
    svals = sin_ref[...]

    @pl.when(j == 0)
    def _init():
        h1 = jnp.dot(prop_ref[...], w1_ref[...],
                     preferred_element_type=jnp.float32) + b1_ref[...]
        h1 = jnp.maximum(h1, 0.0)
        feats = jnp.dot(h1, w2_ref[...],
                        preferred_element_type=jnp.float32) + b2_ref[...]
        feats = jnp.maximum(feats, 0.0)
        feats_s[...] = feats
        for t in range(_T):
            fe_ref[:, t, :] = feats + svals[0, t]
        bestv_s[...] = jnp.full((_B, _T), _NEG, jnp.float32)
        best2_s[...] = jnp.full((_B, _T), _NEG, jnp.float32)
        besti_s[...] = jnp.zeros((_B, _T), jnp.int32)

    feats = feats_s[...]
    wq_q = _bf16_val(wq_ref[...])
    gidx = jax.lax.broadcasted_iota(jnp.int32, (_B, _VT), 1) + j * _VT
    valid = gidx < _V
    for t in range(_T):
        s_t = svals[0, t]
        fet = _bf16_val(feats + s_t)
        logits = jnp.dot(fet, wq_q,
                         preferred_element_type=jnp.float32) + bq_ref[...]
        scores = jnp.where(valid, logits, _NEG)
        m = jnp.max(scores, axis=1, keepdims=True)
        is_max = scores == m
        idx = jnp.min(jnp.where(is_max, gidx, _V), axis=1,
                      keepdims=True)
        m2 = jnp.max(jnp.where(is_max, _NEG, scores), axis=1,
                     keepdims=True)
        prev_v = bestv_s[:, t:t + 1]
        prev_2 = best2_s[:, t:t + 1]
        prev_i = besti_s[:, t:t + 1]
        better = m > prev_v
        new_v = jnp.where(better, m, prev_v)
        new_2 = jnp.maximum(jnp.minimum(m, prev_v), jnp.maximum(m2, prev_2))
        bestv_s[:, t:t + 1] = new_v
        best2_s[:, t:t + 1] = new_2
        besti_s[:, t:t + 1] = jnp.where(better, idx, prev_i)

    @pl.when(j == _NVT - 1)
    def _finish():
        tok_ref[...] = besti_s[...] + _V
        gap_ref[...] = bestv_s[...] - best2_s[...]


def kernel(proprio, W1, b1, W2, b2, Wq, bq, num_tokens):
    del num_tokens
    sin_pos = jnp.sin(jnp.arange(_T).astype(jnp.float32) * 0.1).reshape(1, _T)

    tok, gap, fe = pl.pallas_call(
        _tok_kernel,
        grid=(_NVT,),
        in_specs=[
            pl.BlockSpec((_B, _P), lambda j: (0, 0)),
            pl.BlockSpec((_P, _H), lambda j: (0, 0)),
            pl.BlockSpec((1, _H), lambda j: (0, 0)),
            pl.BlockSpec((_H, _H), lambda j: (0, 0)),
            pl.BlockSpec((1, _H), lambda j: (0, 0)),
            pl.BlockSpec((_H, _VT), lambda j: (0, j)),
            pl.BlockSpec((1, _VT), lambda j: (0, j)),
            pl.BlockSpec((1, _T), lambda j: (0, 0)),
        ],
        out_specs=[
            pl.BlockSpec((_B, _T), lambda j: (0, 0)),
            pl.BlockSpec((_B, _T), lambda j: (0, 0)),
            pl.BlockSpec((_B, _T, _H), lambda j: (0, 0, 0)),
        ],
        out_shape=[
            jax.ShapeDtypeStruct((_B, _T), jnp.int32),
            jax.ShapeDtypeStruct((_B, _T), jnp.float32),
            jax.ShapeDtypeStruct((_B, _T, _H), jnp.float32),
        ],
        scratch_shapes=[
            pltpu.VMEM((_B, _H), jnp.float32),
            pltpu.VMEM((_B, _T), jnp.float32),
            pltpu.VMEM((_B, _T), jnp.float32),
            pltpu.VMEM((_B, _T), jnp.int32),
        ],
    )(proprio, W1, b1.reshape(1, _H), W2, b2.reshape(1, _H),
      Wq, bq.reshape(1, _V), sin_pos)

    del gap
    return tok.astype(jnp.int64), fe


# speedup vs baseline: 1.3412x; 1.3412x over previous
import jax
import jax.numpy as jnp
from jax.experimental import pallas as pl
from jax.experimental.pallas import tpu as pltpu

_B = 1024
_P = 32
_H = 256
_V = 25000
_T = 4
_VT = 2048
_NVT = (_V + _VT - 1) // _VT
_NEG = float("-inf")


def _tok_kernel(prop_ref, w1_ref, b1_ref, w2_ref, b2_ref, wq_ref, bq_ref,
                sin_ref, tok_ref, fe_ref, feats_s, bestv_s, besti_s):
    j = pl.program_id(0)
    svals = sin_ref[...]

    @pl.when(j == 0)
    def _init():
        h1 = jnp.dot(prop_ref[...], w1_ref[...],
                     preferred_element_type=jnp.float32) + b1_ref[...]
        h1 = jnp.maximum(h1, 0.0)
        feats = jnp.dot(h1, w2_ref[...],
                        preferred_element_type=jnp.float32) + b2_ref[...]
        feats = jnp.maximum(feats, 0.0)
        feats_s[...] = feats
        for t in range(_T):
            fe_ref[:, t, :] = feats + svals[0, t]
        bestv_s[...] = jnp.full((_B, _T), _NEG, jnp.float32)
        besti_s[...] = jnp.zeros((_B, _T), jnp.int32)

    feats = feats_s[...]
    wq = wq_ref[...]
    base = jnp.dot(feats, wq,
                   preferred_element_type=jnp.float32) + bq_ref[...]
    csum = jnp.sum(wq, axis=0, keepdims=True)
    gidx = jax.lax.broadcasted_iota(jnp.int32, (_B, _VT), 1) + j * _VT
    valid = gidx < _V
    for t in range(_T):
        s_t = svals[0, t]
        scores = jnp.where(valid, base + s_t * csum, _NEG)
        m = jnp.max(scores, axis=1, keepdims=True)
        idx = jnp.min(jnp.where(scores == m, gidx, _V), axis=1,
                      keepdims=True)
        prev_v = bestv_s[:, t:t + 1]
        prev_i = besti_s[:, t:t + 1]
        better = m > prev_v
        bestv_s[:, t:t + 1] = jnp.where(better, m, prev_v)
        besti_s[:, t:t + 1] = jnp.where(better, idx, prev_i)

    @pl.when(j == _NVT - 1)
    def _finish():
        tok_ref[...] = besti_s[...] + _V


def kernel(proprio, W1, b1, W2, b2, Wq, bq, num_tokens):
    del num_tokens
    sin_pos = jnp.sin(jnp.arange(_T).astype(jnp.float32) * 0.1).reshape(1, _T)

    tok, fe = pl.pallas_call(
        _tok_kernel,
        grid=(_NVT,),
        in_specs=[
            pl.BlockSpec((_B, _P), lambda j: (0, 0)),
            pl.BlockSpec((_P, _H), lambda j: (0, 0)),
            pl.BlockSpec((1, _H), lambda j: (0, 0)),
            pl.BlockSpec((_H, _H), lambda j: (0, 0)),
            pl.BlockSpec((1, _H), lambda j: (0, 0)),
            pl.BlockSpec((_H, _VT), lambda j: (0, j)),
            pl.BlockSpec((1, _VT), lambda j: (0, j)),
            pl.BlockSpec((1, _T), lambda j: (0, 0)),
        ],
        out_specs=[
            pl.BlockSpec((_B, _T), lambda j: (0, 0)),
            pl.BlockSpec((_B, _T, _H), lambda j: (0, 0, 0)),
        ],
        out_shape=[
            jax.ShapeDtypeStruct((_B, _T), jnp.int32),
            jax.ShapeDtypeStruct((_B, _T, _H), jnp.float32),
        ],
        scratch_shapes=[
            pltpu.VMEM((_B, _H), jnp.float32),
            pltpu.VMEM((_B, _T), jnp.float32),
            pltpu.VMEM((_B, _T), jnp.int32),
        ],
    )(proprio, W1, b1.reshape(1, _H), W2, b2.reshape(1, _H),
      Wq, bq.reshape(1, _V), sin_pos)
    return tok.astype(jnp.int64), fe


# speedup vs baseline: 1.6028x; 1.1951x over previous
import jax
import jax.numpy as jnp
from jax.experimental import pallas as pl
from jax.experimental.pallas import tpu as pltpu

_B = 1024
_P = 32
_H = 256
_V = 25000
_T = 4
_VT = 1024
_NVT = (_V + _VT - 1) // _VT
_NEG = float("-inf")


def _tok_kernel(prop_ref, w1_ref, b1_ref, w2_ref, b2_ref, wq_ref, bq_ref,
                sin_ref, tok_ref, fe_ref, feats_s, lanev_s, lanei_s):
    j = pl.program_id(0)
    svals = sin_ref[...]

    @pl.when(j == 0)
    def _init():
        h1 = jnp.dot(prop_ref[...], w1_ref[...],
                     preferred_element_type=jnp.float32) + b1_ref[...]
        h1 = jnp.maximum(h1, 0.0)
        feats = jnp.dot(h1, w2_ref[...],
                        preferred_element_type=jnp.float32) + b2_ref[...]
        feats = jnp.maximum(feats, 0.0)
        feats_s[...] = feats
        for t in range(_T):
            fe_ref[:, t, :] = feats + svals[0, t]
            lanev_s[t][...] = jnp.full((_B, _VT), _NEG, jnp.float32)
            lanei_s[t][...] = jnp.zeros((_B, _VT), jnp.int32)

    feats = feats_s[...]
    wq = wq_ref[...]
    base = jnp.dot(feats, wq,
                   preferred_element_type=jnp.float32) + bq_ref[...]
    csum = jnp.sum(wq, axis=0, keepdims=True)
    lane = jax.lax.broadcasted_iota(jnp.int32, (1, _VT), 1)
    gidx = lane + j * _VT
    valid = gidx < _V
    for t in range(_T):
        s_t = svals[0, t]
        scores = jnp.where(valid, base + s_t * csum, _NEG)
        prev_v = lanev_s[t][...]
        better = scores > prev_v
        lanev_s[t][...] = jnp.where(better, scores, prev_v)
        lanei_s[t][...] = jnp.where(better,
                                    jnp.broadcast_to(gidx, (_B, _VT)),
                                    lanei_s[t][...])

    @pl.when(j == _NVT - 1)
    def _finish():
        for t in range(_T):
            vals = lanev_s[t][...]
            idxs = lanei_s[t][...]
            m = jnp.max(vals, axis=1, keepdims=True)
            pick = jnp.min(jnp.where(vals == m, idxs, _V), axis=1,
                           keepdims=True)
            tok_ref[:, t:t + 1] = pick + _V

def kernel(proprio, W1, b1, W2, b2, Wq, bq, num_tokens):
    del num_tokens
    sin_pos = jnp.sin(jnp.arange(_T).astype(jnp.float32) * 0.1).reshape(1, _T)

    tok, fe = pl.pallas_call(
        _tok_kernel,
        grid=(_NVT,),
        in_specs=[
            pl.BlockSpec((_B, _P), lambda j: (0, 0)),
            pl.BlockSpec((_P, _H), lambda j: (0, 0)),
            pl.BlockSpec((1, _H), lambda j: (0, 0)),
            pl.BlockSpec((_H, _H), lambda j: (0, 0)),
            pl.BlockSpec((1, _H), lambda j: (0, 0)),
            pl.BlockSpec((_H, _VT), lambda j: (0, j)),
            pl.BlockSpec((1, _VT), lambda j: (0, j)),
            pl.BlockSpec((1, _T), lambda j: (0, 0)),
        ],
        out_specs=[
            pl.BlockSpec((_B, _T), lambda j: (0, 0)),
            pl.BlockSpec((_B, _T, _H), lambda j: (0, 0, 0)),
        ],
        out_shape=[
            jax.ShapeDtypeStruct((_B, _T), jnp.int32),
            jax.ShapeDtypeStruct((_B, _T, _H), jnp.float32),
        ],
        scratch_shapes=[
            pltpu.VMEM((_B, _H), jnp.float32),
            [pltpu.VMEM((_B, _VT), jnp.float32) for _ in range(_T)],
            [pltpu.VMEM((_B, _VT), jnp.int32) for _ in range(_T)],
        ],
    )(proprio, W1, b1.reshape(1, _H), W2, b2.reshape(1, _H),
      Wq, bq.reshape(1, _V), sin_pos)
    return tok.astype(jnp.int64), fe
